# baseline (device time: 348893 ns/iter reference)
import jax
import jax.numpy as jnp
from jax import lax
from jax.experimental import pallas as pl
from jax.experimental.pallas import tpu as pltpu

N_DEV = 32
B, SQ, SKV, DM = 2, 512, 512, 768
H_PER = 8
DH = 64
ROWS = B * SQ
CHUNK = ROWS // N_DEV


def _body(x_ref, wq_ref, k_ref, v_ref, wo_ref, out_ref,
          acc_ref, recv_buf, send_sems, recv_sems, credit_sem):
    me = lax.axis_index("i")
    left = jnp.mod(me - 1, N_DEV)
    right = jnp.mod(me + 1, N_DEV)

    barrier = pltpu.get_barrier_semaphore()
    for nbr in (left, right):
        pl.semaphore_signal(barrier, inc=1, device_id=(nbr,),
                            device_id_type=pl.DeviceIdType.MESH)
    pl.semaphore_wait(barrier, 2)

    qi = lax.broadcasted_iota(jnp.int32, (SQ, SKV), 0)
    ki = lax.broadcasted_iota(jnp.int32, (SQ, SKV), 1)
    mask = (jnp.abs(qi - ki) <= 128) | (ki < 32) | (qi < 32)

    for b in range(B):
        xb = x_ref[b]
        acc = jnp.zeros((SQ, DM), jnp.float32)
        for h in range(H_PER):
            q = jnp.dot(xb, wq_ref[h],
                        preferred_element_type=jnp.float32)
            k = k_ref[b, h]
            v = v_ref[b, h]
            s = lax.dot_general(
                q.astype(jnp.bfloat16), k,
                (((1,), (1,)), ((), ())),
                preferred_element_type=jnp.float32,
            ) * 0.125
            s = jnp.where(mask, s, -1e9)
            m = jnp.max(s, axis=-1, keepdims=True)
            w = jnp.exp(s - m)
            w = w / jnp.sum(w, axis=-1, keepdims=True)
            ctx = jnp.dot(w.astype(jnp.bfloat16), v,
                          preferred_element_type=jnp.float32)
            acc = acc + jnp.dot(ctx.astype(jnp.bfloat16), wo_ref[h],
                                preferred_element_type=jnp.float32)
        acc_ref[b * SQ:(b + 1) * SQ, :] = acc

    def hop(g, send_idx, recv_idx):
        slot = lax.rem(g, 2)
        rdma = pltpu.make_async_remote_copy(
            src_ref=acc_ref.at[pl.ds(send_idx * CHUNK, CHUNK), :],
            dst_ref=recv_buf.at[slot],
            send_sem=send_sems.at[slot],
            recv_sem=recv_sems.at[slot],
            device_id=(right,),
            device_id_type=pl.DeviceIdType.MESH,
        )
        rdma.start()
        rdma.wait()
        return slot

    def rs_step(s, carry):
        send_idx = jnp.mod(me - s, N_DEV)
        recv_idx = jnp.mod(me - s - 1, N_DEV)

        @pl.when(s >= 2)
        def _():
            pl.semaphore_wait(credit_sem, 1)

        slot = hop(s, send_idx, recv_idx)
        acc_ref[pl.ds(recv_idx * CHUNK, CHUNK), :] = (
            acc_ref[pl.ds(recv_idx * CHUNK, CHUNK), :] + recv_buf[slot]
        )
        pl.semaphore_signal(credit_sem, inc=1, device_id=(left,),
                            device_id_type=pl.DeviceIdType.MESH)
        return carry

    lax.fori_loop(0, N_DEV - 1, rs_step, 0)

    def ag_step(s, carry):
        g = (N_DEV - 1) + s
        send_idx = jnp.mod(me + 1 - s, N_DEV)
        recv_idx = jnp.mod(me - s, N_DEV)
        pl.semaphore_wait(credit_sem, 1)
        slot = hop(g, send_idx, recv_idx)
        acc_ref[pl.ds(recv_idx * CHUNK, CHUNK), :] = recv_buf[slot]
        pl.semaphore_signal(credit_sem, inc=1, device_id=(left,),
                            device_id_type=pl.DeviceIdType.MESH)
        return carry

    lax.fori_loop(0, N_DEV - 1, ag_step, 0)

    pl.semaphore_wait(credit_sem, 2)

    for b in range(B):
        out_ref[b, :, :] = acc_ref[b * SQ:(b + 1) * SQ, :]


def kernel(x, Wq, K_ext, V_ext, Wo):
    me = lax.axis_index("i")
    xb = x.astype(jnp.bfloat16)
    wq = Wq.reshape(DM, H_PER, DH).transpose(1, 0, 2).astype(jnp.bfloat16)
    wo = Wo.reshape(H_PER, DH, DM).astype(jnp.bfloat16)
    k = lax.dynamic_slice_in_dim(K_ext, me * H_PER, H_PER, axis=2)
    v = lax.dynamic_slice_in_dim(V_ext, me * H_PER, H_PER, axis=2)
    k = k.transpose(0, 2, 1, 3).astype(jnp.bfloat16)
    v = v.transpose(0, 2, 1, 3).astype(jnp.bfloat16)

    return pl.pallas_call(
        _body,
        out_shape=jax.ShapeDtypeStruct((B, SQ, DM), jnp.float32),
        in_specs=[pl.BlockSpec(memory_space=pltpu.VMEM)] * 5,
        out_specs=pl.BlockSpec(memory_space=pltpu.VMEM),
        scratch_shapes=[
            pltpu.VMEM((ROWS, DM), jnp.float32),
            pltpu.VMEM((2, CHUNK, DM), jnp.float32),
            pltpu.SemaphoreType.DMA((2,)),
            pltpu.SemaphoreType.DMA((2,)),
            pltpu.SemaphoreType.REGULAR,
        ],
        compiler_params=pltpu.CompilerParams(collective_id=0),
    )(xb, wq, k, v, wo)


# device time: 183094 ns/iter; 1.9055x vs baseline; 1.9055x over previous
import jax
import jax.numpy as jnp
from jax import lax
from jax.experimental import pallas as pl
from jax.experimental.pallas import tpu as pltpu

N_DEV = 32
B, SQ, SKV, DM = 2, 512, 512, 768
H_PER = 8
DH = 64
ROWS = B * SQ
CHUNK = ROWS // N_DEV

_MESH = pl.DeviceIdType.MESH


def _body(x_ref, wq_ref, k_ref, v_ref, wo_ref, out_ref,
          acc_ref, gather_ref, res_ref,
          send_a, recv_a, send_b, recv_b):
    me = lax.axis_index("i")

    barrier = pltpu.get_barrier_semaphore()

    def _sig(o, c):
        pl.semaphore_signal(barrier, inc=1,
                            device_id=(jnp.mod(me + o, N_DEV),),
                            device_id_type=_MESH)
        return c

    lax.fori_loop(1, N_DEV, _sig, 0)
    pl.semaphore_wait(barrier, N_DEV - 1)

    qi = lax.broadcasted_iota(jnp.int32, (SQ, SKV), 0)
    ki = lax.broadcasted_iota(jnp.int32, (SQ, SKV), 1)
    mask = (jnp.abs(qi - ki) <= 128) | (ki < 32) | (qi < 32)

    for b in range(B):
        xb = x_ref[b]
        acc = jnp.zeros((SQ, DM), jnp.float32)
        for h in range(H_PER):
            q = jnp.dot(xb, wq_ref[h],
                        preferred_element_type=jnp.float32)
            k = k_ref[b, h]
            v = v_ref[b, h]
            s = lax.dot_general(
                q.astype(jnp.bfloat16), k,
                (((1,), (1,)), ((), ())),
                preferred_element_type=jnp.float32,
            ) * 0.125
            s = jnp.where(mask, s, -1e9)
            m = jnp.max(s, axis=-1, keepdims=True)
            w = jnp.exp(s - m)
            w = w / jnp.sum(w, axis=-1, keepdims=True)
            ctx = jnp.dot(w.astype(jnp.bfloat16), v,
                          preferred_element_type=jnp.float32)
            acc = acc + jnp.dot(ctx.astype(jnp.bfloat16), wo_ref[h],
                                preferred_element_type=jnp.float32)
        acc_ref[b * SQ:(b + 1) * SQ, :] = acc

    def _send_a(o, c):
        tgt = jnp.mod(me + o, N_DEV)
        rdma = pltpu.make_async_remote_copy(
            src_ref=acc_ref.at[pl.ds(tgt * CHUNK, CHUNK), :],
            dst_ref=gather_ref.at[me],
            send_sem=send_a.at[tgt],
            recv_sem=recv_a.at[me],
            device_id=(tgt,), device_id_type=_MESH,
        )
        rdma.start()
        return c

    lax.fori_loop(1, N_DEV, _send_a, 0)

    gather_ref[me] = acc_ref[pl.ds(me * CHUNK, CHUNK), :]

    def _wait_a(o, c):
        src = jnp.mod(me + o, N_DEV)
        rdma = pltpu.make_async_remote_copy(
            src_ref=acc_ref.at[pl.ds(0, CHUNK), :],
            dst_ref=gather_ref.at[src],
            send_sem=send_a.at[src],
            recv_sem=recv_a.at[src],
            device_id=(src,), device_id_type=_MESH,
        )
        rdma.wait_recv()
        return c

    lax.fori_loop(1, N_DEV, _wait_a, 0)

    res_ref[pl.ds(me * CHUNK, CHUNK), :] = jnp.sum(
        gather_ref[:, :, :], axis=0)

    def _send_b(o, c):
        tgt = jnp.mod(me + o, N_DEV)
        rdma = pltpu.make_async_remote_copy(
            src_ref=res_ref.at[pl.ds(me * CHUNK, CHUNK), :],
            dst_ref=res_ref.at[pl.ds(me * CHUNK, CHUNK), :],
            send_sem=send_b.at[tgt],
            recv_sem=recv_b.at[me],
            device_id=(tgt,), device_id_type=_MESH,
        )
        rdma.start()
        return c

    lax.fori_loop(1, N_DEV, _send_b, 0)

    def _wait_b(o, c):
        src = jnp.mod(me + o, N_DEV)
        rdma = pltpu.make_async_remote_copy(
            src_ref=res_ref.at[pl.ds(0, CHUNK), :],
            dst_ref=res_ref.at[pl.ds(src * CHUNK, CHUNK), :],
            send_sem=send_b.at[src],
            recv_sem=recv_b.at[src],
            device_id=(src,), device_id_type=_MESH,
        )
        rdma.wait_recv()
        return c

    lax.fori_loop(1, N_DEV, _wait_b, 0)

    def _drain(o, c):
        tgt = jnp.mod(me + o, N_DEV)
        for sem_arr, src in ((send_a, acc_ref), (send_b, res_ref)):
            rdma = pltpu.make_async_remote_copy(
                src_ref=src.at[pl.ds(0, CHUNK), :],
                dst_ref=gather_ref.at[0],
                send_sem=sem_arr.at[tgt],
                recv_sem=recv_a.at[0],
                device_id=(tgt,), device_id_type=_MESH,
            )
            rdma.wait_send()
        return c

    lax.fori_loop(1, N_DEV, _drain, 0)

    for b in range(B):
        out_ref[b, :, :] = res_ref[b * SQ:(b + 1) * SQ, :]


def kernel(x, Wq, K_ext, V_ext, Wo):
    me = lax.axis_index("i")
    xb = x.astype(jnp.bfloat16)
    wq = Wq.reshape(DM, H_PER, DH).transpose(1, 0, 2).astype(jnp.bfloat16)
    wo = Wo.reshape(H_PER, DH, DM).astype(jnp.bfloat16)
    k = lax.dynamic_slice_in_dim(K_ext, me * H_PER, H_PER, axis=2)
    v = lax.dynamic_slice_in_dim(V_ext, me * H_PER, H_PER, axis=2)
    k = k.transpose(0, 2, 1, 3).astype(jnp.bfloat16)
    v = v.transpose(0, 2, 1, 3).astype(jnp.bfloat16)

    return pl.pallas_call(
        _body,
        out_shape=jax.ShapeDtypeStruct((B, SQ, DM), jnp.float32),
        in_specs=[pl.BlockSpec(memory_space=pltpu.VMEM)] * 5,
        out_specs=pl.BlockSpec(memory_space=pltpu.VMEM),
        scratch_shapes=[
            pltpu.VMEM((ROWS, DM), jnp.float32),
            pltpu.VMEM((N_DEV, CHUNK, DM), jnp.float32),
            pltpu.VMEM((ROWS, DM), jnp.float32),
            pltpu.SemaphoreType.DMA((N_DEV,)),
            pltpu.SemaphoreType.DMA((N_DEV,)),
            pltpu.SemaphoreType.DMA((N_DEV,)),
            pltpu.SemaphoreType.DMA((N_DEV,)),
        ],
        compiler_params=pltpu.CompilerParams(collective_id=0),
    )(xb, wq, k, v, wo)
